# Initial kernel scaffold; baseline (speedup 1.0000x reference)
import jax
import jax.numpy as jnp
from jax.experimental import pallas as pl


def kernel(x, fc1_w, fc1_b, fc2_w, fc2_b, w_gate, exp_w1, exp_b1, exp_w2, exp_b2, tower_w, tower_b):
    raise NotImplementedError("write your pallas kernel here")


# speedup vs baseline: 34.7323x; 34.7323x over previous
import functools

import jax
import jax.numpy as jnp
from jax.experimental import pallas as pl

B = 32
NC = 15
NT = 4
NE = 8
TOPK = 4
R = B * NC


def _stage_a(x_ref, w1_ref, b1_ref, w2_ref, b2_ref, wg_ref, h_ref, l_ref):
    a = jnp.maximum(
        jnp.dot(x_ref[...], w1_ref[...], preferred_element_type=jnp.float32)
        + b1_ref[...], 0.0)
    h = jnp.dot(a, w2_ref[...], preferred_element_type=jnp.float32) + b2_ref[...]
    h_ref[...] = h
    l_ref[...] = jnp.dot(h, wg_ref[...], preferred_element_type=jnp.float32)


def _stage_b(h_ref, w1_ref, b1_ref, w2_ref, b2_ref, twt_ref, s_ref):
    a = jnp.maximum(
        jnp.dot(h_ref[...], w1_ref[0], preferred_element_type=jnp.float32)
        + b1_ref[0], 0.0)
    vt = jnp.dot(w2_ref[0], twt_ref[...], preferred_element_type=jnp.float32)
    sb = jnp.dot(b2_ref[0], twt_ref[...], preferred_element_type=jnp.float32)
    s_ref[0] = jnp.dot(a, vt, preferred_element_type=jnp.float32) + sb


def _stage_c(l_ref, s_ref, tb_ref, out_ref, aux_ref):
    row_i = jax.lax.broadcasted_iota(jnp.int32, (NC, R), 0)
    col_i = jax.lax.broadcasted_iota(jnp.int32, (NC, R), 1)
    sel = (col_i // B == row_i).astype(jnp.float32)
    lane_e = jax.lax.broadcasted_iota(jnp.int32, (R, NE), 1)
    total = jnp.float32(0.0)
    for t in range(NT):
        lt = l_ref[t]
        rank = jnp.zeros((R, NE), dtype=jnp.int32)
        for j in range(NE):
            lj = lt[:, j:j + 1]
            gt = (lj > lt) | ((lj == lt) & (j < lane_e))
            rank = rank + gt.astype(jnp.int32)
        mask = rank < TOPK
        maskf = mask.astype(jnp.float32)
        mx = jnp.max(lt, axis=1, keepdims=True)
        ex = jnp.exp(lt - mx) * maskf
        gates = ex / jnp.sum(ex, axis=1, keepdims=True)
        importance = jnp.dot(sel, gates, preferred_element_type=jnp.float32)
        load = jnp.dot(sel, maskf, preferred_element_type=jnp.float32)

        def _cv2(v):
            m = jnp.sum(v, axis=1, keepdims=True) / NE
            var = jnp.sum((v - m) ** 2, axis=1, keepdims=True) / (NE - 1)
            return var / (m * m + 1e-10)

        total = total + jnp.sum(_cv2(importance) + _cv2(load)) * 1e-2
        score = jnp.sum(gates * s_ref[t], axis=1) + tb_ref[0, t]
        out_ref[t, :] = 1.0 / (1.0 + jnp.exp(-score))
    aux_ref[...] = jnp.reshape(total, (1, 1))


def kernel(x, fc1_w, fc1_b, fc2_w, fc2_b, w_gate, exp_w1, exp_b1, exp_w2,
           exp_b2, tower_w, tower_b):
    xr = x.transpose(1, 0, 2).reshape(R, x.shape[2])
    wg = w_gate.transpose(1, 0, 2).reshape(w_gate.shape[1], NT * NE)
    h, l_flat = pl.pallas_call(
        _stage_a,
        out_shape=(
            jax.ShapeDtypeStruct((R, fc2_w.shape[1]), jnp.float32),
            jax.ShapeDtypeStruct((R, NT * NE), jnp.float32),
        ),
    )(xr, fc1_w, fc1_b.reshape(1, -1), fc2_w, fc2_b.reshape(1, -1), wg)

    twt = tower_w.T
    s_raw = pl.pallas_call(
        _stage_b,
        grid=(NE,),
        in_specs=[
            pl.BlockSpec((R, h.shape[1]), lambda e: (0, 0)),
            pl.BlockSpec((1,) + exp_w1.shape[1:], lambda e: (e, 0, 0)),
            pl.BlockSpec((1, 1, exp_b1.shape[1]), lambda e: (e, 0, 0)),
            pl.BlockSpec((1,) + exp_w2.shape[1:], lambda e: (e, 0, 0)),
            pl.BlockSpec((1, 1, exp_b2.shape[1]), lambda e: (e, 0, 0)),
            pl.BlockSpec(twt.shape, lambda e: (0, 0)),
        ],
        out_specs=pl.BlockSpec((1, R, NT), lambda e: (e, 0, 0)),
        out_shape=jax.ShapeDtypeStruct((NE, R, NT), jnp.float32),
    )(h, exp_w1, exp_b1.reshape(NE, 1, -1), exp_w2, exp_b2.reshape(NE, 1, -1),
      twt)

    lt = l_flat.reshape(R, NT, NE).transpose(1, 0, 2)
    st = s_raw.transpose(2, 1, 0)
    scores, aux = pl.pallas_call(
        _stage_c,
        out_shape=(
            jax.ShapeDtypeStruct((NT, R), jnp.float32),
            jax.ShapeDtypeStruct((1, 1), jnp.float32),
        ),
    )(lt, st, tower_b.reshape(1, NT))

    pred = scores.reshape(NT, NC, B).transpose(2, 1, 0)
    return pred, aux[0, 0]


# speedup vs baseline: 37.5346x; 1.0807x over previous
import jax
import jax.numpy as jnp
from jax.experimental import pallas as pl

B = 32
NC = 15
NT = 4
NE = 8
TOPK = 4
R = B * NC


def _stage_a(x_ref, w1_ref, b1_ref, w2_ref, b2_ref, wg_ref, h_ref, l_ref):
    a = jnp.maximum(
        jnp.dot(x_ref[...], w1_ref[...], preferred_element_type=jnp.float32)
        + b1_ref[...], 0.0)
    h = jnp.dot(a, w2_ref[...], preferred_element_type=jnp.float32) + b2_ref[...]
    h_ref[...] = h
    l_ref[...] = jnp.dot(h, wg_ref[...], preferred_element_type=jnp.float32)


def _stage_b(h_ref, w1_ref, b1_ref, w2_ref, b2_ref, twt_ref, s_ref):
    a = jnp.maximum(
        jnp.dot(h_ref[...].astype(jnp.bfloat16), w1_ref[0].astype(jnp.bfloat16),
                preferred_element_type=jnp.float32) + b1_ref[0], 0.0)
    vt = jnp.dot(w2_ref[0].astype(jnp.bfloat16), twt_ref[...].astype(jnp.bfloat16),
                 preferred_element_type=jnp.float32)
    sb = jnp.dot(b2_ref[0], twt_ref[...], preferred_element_type=jnp.float32)
    s_ref[0] = jnp.dot(a.astype(jnp.bfloat16), vt.astype(jnp.bfloat16),
                       preferred_element_type=jnp.float32) + sb


def _stage_c(l_ref, s_ref, tb_ref, out_ref, aux_ref):
    f32 = jnp.float32
    L = l_ref[...]
    c_row = jax.lax.broadcasted_iota(jnp.int32, (NT * NE, NT * NE), 0)
    c_col = jax.lax.broadcasted_iota(jnp.int32, (NT * NE, NT * NE), 1)
    grp8 = (c_row // NE == c_col // NE).astype(f32)
    lane = jax.lax.broadcasted_iota(jnp.int32, (R, NT * NE), 1)
    e_mod = lane % NE
    rank = jnp.zeros((R, NT * NE), dtype=jnp.int32)
    for j in range(NE):
        pj = (c_row == (c_col // NE) * NE + j).astype(f32)
        lj = jnp.dot(L, pj, preferred_element_type=f32)
        gt = (lj > L) | ((lj == L) & (j < e_mod))
        rank = rank + gt.astype(jnp.int32)
    maskf = (rank < TOPK).astype(f32)
    mx = jnp.max(L, axis=1, keepdims=True)
    ex = jnp.exp(L - mx) * maskf
    den = jnp.dot(ex, grp8, preferred_element_type=f32)
    gates = ex / den
    r_row = jax.lax.broadcasted_iota(jnp.int32, (NC, R), 0)
    r_col = jax.lax.broadcasted_iota(jnp.int32, (NC, R), 1)
    sel = (r_col % NC == r_row).astype(f32)
    def _cv2_terms(v):
        m = jnp.dot(v, grp8, preferred_element_type=f32) / NE
        d = v - m
        var = jnp.dot(d * d, grp8, preferred_element_type=f32) / (NE - 1)
        return var / (m * m + 1e-10)

    imp = jnp.dot(sel, gates, preferred_element_type=f32)
    load = jnp.dot(sel, maskf, preferred_element_type=f32)
    total = (jnp.sum(_cv2_terms(imp)) + jnp.sum(_cv2_terms(load))) / NE * 1e-2
    m_col = jax.lax.broadcasted_iota(jnp.int32, (NT * NE, NT), 1)
    msel = (jax.lax.broadcasted_iota(jnp.int32, (NT * NE, NT), 0) // NE
            == m_col).astype(f32)
    score = jnp.dot(gates * s_ref[...], msel, preferred_element_type=f32)
    score = score + tb_ref[...]
    out_ref[...] = 1.0 / (1.0 + jnp.exp(-score))
    aux_ref[...] = jnp.reshape(total, (1, 1))


def kernel(x, fc1_w, fc1_b, fc2_w, fc2_b, w_gate, exp_w1, exp_b1, exp_w2,
           exp_b2, tower_w, tower_b):
    xr = x.reshape(R, x.shape[2])
    wg = w_gate.transpose(1, 0, 2).reshape(w_gate.shape[1], NT * NE)
    h, l_flat = pl.pallas_call(
        _stage_a,
        out_shape=(
            jax.ShapeDtypeStruct((R, fc2_w.shape[1]), jnp.float32),
            jax.ShapeDtypeStruct((R, NT * NE), jnp.float32),
        ),
    )(xr, fc1_w, fc1_b.reshape(1, -1), fc2_w, fc2_b.reshape(1, -1), wg)

    twt = tower_w.T
    s_raw = pl.pallas_call(
        _stage_b,
        grid=(NE,),
        in_specs=[
            pl.BlockSpec((R, h.shape[1]), lambda e: (0, 0)),
            pl.BlockSpec((1,) + exp_w1.shape[1:], lambda e: (e, 0, 0)),
            pl.BlockSpec((1, 1, exp_b1.shape[1]), lambda e: (e, 0, 0)),
            pl.BlockSpec((1,) + exp_w2.shape[1:], lambda e: (e, 0, 0)),
            pl.BlockSpec((1, 1, exp_b2.shape[1]), lambda e: (e, 0, 0)),
            pl.BlockSpec(twt.shape, lambda e: (0, 0)),
        ],
        out_specs=pl.BlockSpec((1, R, NT), lambda e: (e, 0, 0)),
        out_shape=jax.ShapeDtypeStruct((NE, R, NT), jnp.float32),
    )(h, exp_w1, exp_b1.reshape(NE, 1, -1), exp_w2, exp_b2.reshape(NE, 1, -1),
      twt)

    st = s_raw.transpose(1, 2, 0).reshape(R, NT * NE)
    scores, aux = pl.pallas_call(
        _stage_c,
        out_shape=(
            jax.ShapeDtypeStruct((R, NT), jnp.float32),
            jax.ShapeDtypeStruct((1, 1), jnp.float32),
        ),
    )(l_flat, st, tower_b.reshape(1, NT))

    return scores.reshape(B, NC, NT), aux[0, 0]
